# baseline (speedup 1.0000x reference)
import functools

import jax
import jax.numpy as jnp
from jax import lax
from jax.experimental import pallas as pl
from jax.experimental.pallas import tpu as pltpu
from jax.experimental.pallas import tpu_sc as plsc

B = 16384
D = 64
_INFO = plsc.get_sparse_core_info()
NC = _INFO.num_cores
NS = _INFO.num_subcores
NW = NC * NS
B_PER_W = B // NW
IDX_CHUNK = 128
CHUNKS = B_PER_W // IDX_CHUNK


def _sc_gather_body(u_ids, i_ids, u_tab, i_tab, u_out, i_out,
                    uidx_v, iidx_v, urows_v, irows_v, sem):
    wid = lax.axis_index("s") * NC + lax.axis_index("c")
    row0 = wid * CHUNKS
    pltpu.sync_copy(u_ids.at[pl.ds(row0, CHUNKS)], uidx_v)
    pltpu.sync_copy(i_ids.at[pl.ds(row0, CHUNKS)], iidx_v)
    copies = []
    for j in range(CHUNKS):
        copies.append(pltpu.async_copy(
            u_tab.at[uidx_v.at[j]], urows_v.at[pl.ds(j * IDX_CHUNK, IDX_CHUNK)], sem))
        copies.append(pltpu.async_copy(
            i_tab.at[iidx_v.at[j]], irows_v.at[pl.ds(j * IDX_CHUNK, IDX_CHUNK)], sem))
    for c in copies:
        c.wait()
    base = wid * B_PER_W
    pltpu.sync_copy(urows_v, u_out.at[pl.ds(base, B_PER_W)])
    pltpu.sync_copy(irows_v, i_out.at[pl.ds(base, B_PER_W)])


def _make_sc_gather():
    mesh = plsc.VectorSubcoreMesh(core_axis_name="c", subcore_axis_name="s")
    return pl.kernel(
        _sc_gather_body,
        mesh=mesh,
        out_type=[
            jax.ShapeDtypeStruct((B, D), jnp.float32),
            jax.ShapeDtypeStruct((B, D), jnp.float32),
        ],
        scratch_types=[
            pltpu.VMEM((CHUNKS, IDX_CHUNK), jnp.int32),
            pltpu.VMEM((CHUNKS, IDX_CHUNK), jnp.int32),
            pltpu.VMEM((B_PER_W, D), jnp.float32),
            pltpu.VMEM((B_PER_W, D), jnp.float32),
            pltpu.SemaphoreType.DMA,
        ],
        compiler_params=pltpu.CompilerParams(use_tc_tiling_on_sc=False),
    )


_BB = 2048


def _mlp_body(u_ref, i_ref, w1u_ref, w1i_ref, b1_ref, w2_ref, b2_ref,
              w3_ref, b3_ref, w4_ref, b4_ref, out_ref):
    h = jnp.dot(u_ref[...], w1u_ref[...], preferred_element_type=jnp.float32)
    h = h + jnp.dot(i_ref[...], w1i_ref[...], preferred_element_type=jnp.float32)
    h = jnp.maximum(h + b1_ref[...], 0.0)
    h = jnp.maximum(jnp.dot(h, w2_ref[...], preferred_element_type=jnp.float32)
                    + b2_ref[...], 0.0)
    h = jnp.maximum(jnp.dot(h, w3_ref[...], preferred_element_type=jnp.float32)
                    + b3_ref[...], 0.0)
    out_ref[...] = (jnp.sum(h * w4_ref[...], axis=1, keepdims=True)
                    + b4_ref[...])


def _mlp(u_emb, i_emb, W1, b1, W2, b2, W3, b3, W4, b4):
    w1u = W1[:, :D].T
    w1i = W1[:, D:].T
    w2 = W2.T
    w3 = W3.T
    w4 = W4
    grid = B // _BB
    full = lambda s: pl.BlockSpec(s, lambda i: (0, 0))
    out = pl.pallas_call(
        _mlp_body,
        grid=(grid,),
        in_specs=[
            pl.BlockSpec((_BB, D), lambda i: (i, 0)),
            pl.BlockSpec((_BB, D), lambda i: (i, 0)),
            full(w1u.shape), full(w1i.shape), full((1, 32)),
            full(w2.shape), full((1, 16)),
            full(w3.shape), full((1, 8)),
            full(w4.shape), full((1, 1)),
        ],
        out_specs=pl.BlockSpec((_BB, 1), lambda i: (i, 0)),
        out_shape=jax.ShapeDtypeStruct((B, 1), jnp.float32),
    )(u_emb, i_emb, w1u, w1i, b1.reshape(1, 32), w2, b2.reshape(1, 16),
      w3, b3.reshape(1, 8), w4, b4.reshape(1, 1))
    return out.reshape(-1)


def kernel(user_ids, item_ids, user_table, item_table,
           W1, b1, W2, b2, W3, b3, W4, b4):
    u_ids = user_ids.astype(jnp.int32).reshape(NW * CHUNKS, IDX_CHUNK)
    i_ids = item_ids.astype(jnp.int32).reshape(NW * CHUNKS, IDX_CHUNK)
    u_emb, i_emb = _make_sc_gather()(u_ids, i_ids, user_table, item_table)
    return _mlp(u_emb, i_emb, W1, b1, W2, b2, W3, b3, W4, b4)


# speedup vs baseline: 1.0038x; 1.0038x over previous
import jax
import jax.numpy as jnp
from jax import lax
from jax.experimental import pallas as pl
from jax.experimental.pallas import tpu as pltpu
from jax.experimental.pallas import tpu_sc as plsc

B = 16384
D = 64
PACK = 2 * D
_INFO = plsc.get_sparse_core_info()
NC = _INFO.num_cores
NS = _INFO.num_subcores
NW = NC * NS
B_PER_W = B // NW
IDX_CHUNK = 128
CHUNKS = B_PER_W // IDX_CHUNK


def _sc_gather_body(u_ids, i_ids, u_tab, i_tab, u_out, i_out,
                    idx_v, rows_v, sem):
    wid = lax.axis_index("s") * NC + lax.axis_index("c")
    row0 = wid * CHUNKS
    base = wid * B_PER_W
    for tab, ids, out in ((u_tab, u_ids, u_out), (i_tab, i_ids, i_out)):
        pltpu.sync_copy(ids.at[pl.ds(row0, CHUNKS)], idx_v)
        copies = []
        for j in range(CHUNKS):
            copies.append(pltpu.async_copy(
                tab.at[idx_v.at[j]],
                rows_v.at[pl.ds(j * IDX_CHUNK, IDX_CHUNK)], sem))
        for c in copies:
            c.wait()
        pltpu.sync_copy(rows_v, out.at[pl.ds(base, B_PER_W)])


def _make_sc_gather():
    mesh = plsc.VectorSubcoreMesh(core_axis_name="c", subcore_axis_name="s")
    return pl.kernel(
        _sc_gather_body,
        mesh=mesh,
        out_type=[
            jax.ShapeDtypeStruct((B, PACK), jnp.float32),
            jax.ShapeDtypeStruct((B, PACK), jnp.float32),
        ],
        scratch_types=[
            pltpu.VMEM((CHUNKS, IDX_CHUNK), jnp.int32),
            pltpu.VMEM((B_PER_W, PACK), jnp.float32),
            pltpu.SemaphoreType.DMA,
        ],
    )


_BB = 2048


def _mlp_body(u_ref, i_ref, pu_ref, pi_ref, w1ue_ref, w1uo_ref, w1ie_ref,
              w1io_ref, b1_ref, w2_ref, b2_ref, w3_ref, b3_ref, w4_ref,
              b4_ref, out_ref):
    u = u_ref[...]
    i = i_ref[...]
    hu = jnp.where(
        pu_ref[...] > 0,
        jnp.dot(u, w1uo_ref[...], preferred_element_type=jnp.float32),
        jnp.dot(u, w1ue_ref[...], preferred_element_type=jnp.float32))
    hi = jnp.where(
        pi_ref[...] > 0,
        jnp.dot(i, w1io_ref[...], preferred_element_type=jnp.float32),
        jnp.dot(i, w1ie_ref[...], preferred_element_type=jnp.float32))
    h = jnp.maximum(hu + hi + b1_ref[...], 0.0)
    h = jnp.maximum(jnp.dot(h, w2_ref[...], preferred_element_type=jnp.float32)
                    + b2_ref[...], 0.0)
    h = jnp.maximum(jnp.dot(h, w3_ref[...], preferred_element_type=jnp.float32)
                    + b3_ref[...], 0.0)
    out_ref[...] = (jnp.sum(h * w4_ref[...], axis=1, keepdims=True)
                    + b4_ref[...])


def _mlp(u_pack, i_pack, pu, pi, W1, b1, W2, b2, W3, b3, W4, b4):
    z = jnp.zeros((D, 32), jnp.float32)
    w1u = W1[:, :D].T
    w1i = W1[:, D:].T
    w1ue = jnp.concatenate([w1u, z], axis=0)
    w1uo = jnp.concatenate([z, w1u], axis=0)
    w1ie = jnp.concatenate([w1i, z], axis=0)
    w1io = jnp.concatenate([z, w1i], axis=0)
    w2 = W2.T
    w3 = W3.T
    w4 = W4
    grid = B // _BB
    full = lambda s: pl.BlockSpec(s, lambda i: (0, 0))
    blk = lambda w: pl.BlockSpec((_BB, w), lambda i: (i, 0))
    out = pl.pallas_call(
        _mlp_body,
        grid=(grid,),
        in_specs=[
            blk(PACK), blk(PACK), blk(1), blk(1),
            full((PACK, 32)), full((PACK, 32)), full((PACK, 32)),
            full((PACK, 32)), full((1, 32)),
            full((32, 16)), full((1, 16)),
            full((16, 8)), full((1, 8)),
            full((1, 8)), full((1, 1)),
        ],
        out_specs=pl.BlockSpec((_BB, 1), lambda i: (i, 0)),
        out_shape=jax.ShapeDtypeStruct((B, 1), jnp.float32),
    )(u_pack, i_pack, pu, pi, w1ue, w1uo, w1ie, w1io, b1.reshape(1, 32),
      w2, b2.reshape(1, 16), w3, b3.reshape(1, 8), w4, b4.reshape(1, 1))
    return out.reshape(-1)


def kernel(user_ids, item_ids, user_table, item_table,
           W1, b1, W2, b2, W3, b3, W4, b4):
    uid = user_ids.astype(jnp.int32)
    iid = item_ids.astype(jnp.int32)
    u_ids = (uid // 2).reshape(NW * CHUNKS, IDX_CHUNK)
    i_ids = (iid // 2).reshape(NW * CHUNKS, IDX_CHUNK)
    pu = (uid % 2).astype(jnp.float32).reshape(B, 1)
    pi = (iid % 2).astype(jnp.float32).reshape(B, 1)
    u_tab = user_table.reshape(-1, PACK)
    i_tab = item_table.reshape(-1, PACK)
    u_pack, i_pack = _make_sc_gather()(u_ids, i_ids, u_tab, i_tab)
    return _mlp(u_pack, i_pack, pu, pi, W1, b1, W2, b2, W3, b3, W4, b4)


# speedup vs baseline: 1.6521x; 1.6458x over previous
import jax
import jax.numpy as jnp
from jax import lax
from jax.experimental import pallas as pl
from jax.experimental.pallas import tpu as pltpu
from jax.experimental.pallas import tpu_sc as plsc

B = 16384
D = 64
PACK = 2 * D
NROWS = 1000000
_INFO = plsc.get_sparse_core_info()
NC = _INFO.num_cores
NS = _INFO.num_subcores
NW = NC * NS
B_PER_W = B // NW
IDX_CHUNK = 128
CHUNKS = B_PER_W // IDX_CHUNK

_TC = 2048


def _combine_body(u_ref, i_ref, c_ref):
    c_ref[...] = jnp.concatenate([u_ref[...].T, i_ref[...].T], axis=1)


def _combine_tables(u_tabT, i_tabT):
    grid = (NROWS + _TC - 1) // _TC
    return pl.pallas_call(
        _combine_body,
        grid=(grid,),
        in_specs=[
            pl.BlockSpec((D, _TC), lambda g: (0, g)),
            pl.BlockSpec((D, _TC), lambda g: (0, g)),
        ],
        out_specs=pl.BlockSpec((_TC, PACK), lambda g: (g, 0)),
        out_shape=jax.ShapeDtypeStruct((NROWS, PACK), jnp.float32),
    )(u_tabT, i_tabT)


def _sc_gather_body(u_ids, i_ids, c_tab, u_out, i_out, idx_v, rows_v, sem):
    wid = lax.axis_index("s") * NC + lax.axis_index("c")
    row0 = wid * CHUNKS
    base = wid * B_PER_W
    for ids, out in ((u_ids, u_out), (i_ids, i_out)):
        pltpu.sync_copy(ids.at[pl.ds(row0, CHUNKS)], idx_v)
        copies = []
        for j in range(CHUNKS):
            copies.append(pltpu.async_copy(
                c_tab.at[idx_v.at[j]],
                rows_v.at[pl.ds(j * IDX_CHUNK, IDX_CHUNK)], sem))
        for c in copies:
            c.wait()
        pltpu.sync_copy(rows_v, out.at[pl.ds(base, B_PER_W)])


def _make_sc_gather():
    mesh = plsc.VectorSubcoreMesh(core_axis_name="c", subcore_axis_name="s")
    return pl.kernel(
        _sc_gather_body,
        mesh=mesh,
        out_type=[
            jax.ShapeDtypeStruct((B, PACK), jnp.float32),
            jax.ShapeDtypeStruct((B, PACK), jnp.float32),
        ],
        scratch_types=[
            pltpu.VMEM((CHUNKS, IDX_CHUNK), jnp.int32),
            pltpu.VMEM((B_PER_W, PACK), jnp.float32),
            pltpu.SemaphoreType.DMA,
        ],
    )


_BB = 2048


def _mlp_body(u_ref, i_ref, w1u_ref, w1i_ref, b1_ref, w2_ref, b2_ref,
              w3_ref, b3_ref, w4_ref, b4_ref, out_ref):
    h = jnp.dot(u_ref[...], w1u_ref[...], preferred_element_type=jnp.float32)
    h = h + jnp.dot(i_ref[...], w1i_ref[...],
                    preferred_element_type=jnp.float32)
    h = jnp.maximum(h + b1_ref[...], 0.0)
    h = jnp.maximum(jnp.dot(h, w2_ref[...], preferred_element_type=jnp.float32)
                    + b2_ref[...], 0.0)
    h = jnp.maximum(jnp.dot(h, w3_ref[...], preferred_element_type=jnp.float32)
                    + b3_ref[...], 0.0)
    out_ref[...] = (jnp.sum(h * w4_ref[...], axis=1, keepdims=True)
                    + b4_ref[...])


def _mlp(gu, gi, W1, b1, W2, b2, W3, b3, W4, b4):
    z = jnp.zeros((D, 32), jnp.float32)
    w1u = jnp.concatenate([W1[:, :D].T, z], axis=0)
    w1i = jnp.concatenate([z, W1[:, D:].T], axis=0)
    grid = B // _BB
    full = lambda s: pl.BlockSpec(s, lambda i: (0, 0))
    blk = lambda w: pl.BlockSpec((_BB, w), lambda i: (i, 0))
    out = pl.pallas_call(
        _mlp_body,
        grid=(grid,),
        in_specs=[
            blk(PACK), blk(PACK),
            full((PACK, 32)), full((PACK, 32)), full((1, 32)),
            full((32, 16)), full((1, 16)),
            full((16, 8)), full((1, 8)),
            full((1, 8)), full((1, 1)),
        ],
        out_specs=pl.BlockSpec((_BB, 1), lambda i: (i, 0)),
        out_shape=jax.ShapeDtypeStruct((B, 1), jnp.float32),
    )(gu, gi, w1u, w1i, b1.reshape(1, 32),
      W2.T, b2.reshape(1, 16), W3.T, b3.reshape(1, 8), W4, b4.reshape(1, 1))
    return out.reshape(-1)


def kernel(user_ids, item_ids, user_table, item_table,
           W1, b1, W2, b2, W3, b3, W4, b4):
    u_ids = user_ids.astype(jnp.int32).reshape(NW * CHUNKS, IDX_CHUNK)
    i_ids = item_ids.astype(jnp.int32).reshape(NW * CHUNKS, IDX_CHUNK)
    c_tab = _combine_tables(user_table.T, item_table.T)
    gu, gi = _make_sc_gather()(u_ids, i_ids, c_tab)
    return _mlp(gu, gi, W1, b1, W2, b2, W3, b3, W4, b4)


# speedup vs baseline: 2.0581x; 1.2458x over previous
import jax
import jax.numpy as jnp
from jax import lax
from jax.experimental import pallas as pl
from jax.experimental.pallas import tpu as pltpu
from jax.experimental.pallas import tpu_sc as plsc

B = 16384
D = 64
PACK = 2 * D
NROWS = 1000000
_INFO = plsc.get_sparse_core_info()
NC = _INFO.num_cores
NS = _INFO.num_subcores
NW = NC * NS
B_PER_W = B // NW
IDX_CHUNK = 128
CHUNKS = B_PER_W // IDX_CHUNK

_TC = 4096


def _combine_body(u_ref, i_ref, c_ref):
    c_ref[...] = jnp.concatenate([u_ref[...].T, i_ref[...].T], axis=1)


def _combine_tables(u_tabT, i_tabT):
    grid = (NROWS + _TC - 1) // _TC
    return pl.pallas_call(
        _combine_body,
        grid=(grid,),
        in_specs=[
            pl.BlockSpec((D, _TC), lambda g: (0, g)),
            pl.BlockSpec((D, _TC), lambda g: (0, g)),
        ],
        out_specs=pl.BlockSpec((_TC, PACK), lambda g: (g, 0)),
        out_shape=jax.ShapeDtypeStruct((NROWS, PACK), jnp.float32),
    )(u_tabT, i_tabT)


def _sc_gather_body(u_ids, i_ids, c_tab, u_out, i_out, idx_v, rows_v, sem):
    wid = lax.axis_index("s") * NC + lax.axis_index("c")
    row0 = wid * CHUNKS
    base = wid * B_PER_W
    for ids, out in ((u_ids, u_out), (i_ids, i_out)):
        pltpu.sync_copy(ids.at[pl.ds(row0, CHUNKS)], idx_v)
        copies = []
        for j in range(CHUNKS):
            copies.append(pltpu.async_copy(
                c_tab.at[idx_v.at[j]],
                rows_v.at[pl.ds(j * IDX_CHUNK, IDX_CHUNK)], sem))
        for c in copies:
            c.wait()
        pltpu.sync_copy(rows_v, out.at[pl.ds(base, B_PER_W)])


def _make_sc_gather():
    mesh = plsc.VectorSubcoreMesh(core_axis_name="c", subcore_axis_name="s")
    return pl.kernel(
        _sc_gather_body,
        mesh=mesh,
        out_type=[
            jax.ShapeDtypeStruct((B, PACK), jnp.float32),
            jax.ShapeDtypeStruct((B, PACK), jnp.float32),
        ],
        scratch_types=[
            pltpu.VMEM((CHUNKS, IDX_CHUNK), jnp.int32),
            pltpu.VMEM((B_PER_W, PACK), jnp.float32),
            pltpu.SemaphoreType.DMA,
        ],
    )


_BB = 2048


def _mlp_body(u_ref, i_ref, w1u_ref, w1i_ref, b1_ref, w2_ref, b2_ref,
              w3_ref, b3_ref, w4_ref, b4_ref, out_ref):
    h = jnp.dot(u_ref[...], w1u_ref[...], preferred_element_type=jnp.float32)
    h = h + jnp.dot(i_ref[...], w1i_ref[...],
                    preferred_element_type=jnp.float32)
    h = jnp.maximum(h + b1_ref[...], 0.0)
    h = jnp.maximum(jnp.dot(h, w2_ref[...], preferred_element_type=jnp.float32)
                    + b2_ref[...], 0.0)
    h = jnp.maximum(jnp.dot(h, w3_ref[...], preferred_element_type=jnp.float32)
                    + b3_ref[...], 0.0)
    out_ref[...] = (jnp.sum(h * w4_ref[...], axis=1, keepdims=True)
                    + b4_ref[...])


def _mlp(gu, gi, W1, b1, W2, b2, W3, b3, W4, b4):
    z = jnp.zeros((D, 32), jnp.float32)
    w1u = jnp.concatenate([W1[:, :D].T, z], axis=0)
    w1i = jnp.concatenate([z, W1[:, D:].T], axis=0)
    grid = B // _BB
    full = lambda s: pl.BlockSpec(s, lambda i: (0, 0))
    blk = lambda w: pl.BlockSpec((_BB, w), lambda i: (i, 0))
    out = pl.pallas_call(
        _mlp_body,
        grid=(grid,),
        in_specs=[
            blk(PACK), blk(PACK),
            full((PACK, 32)), full((PACK, 32)), full((1, 32)),
            full((32, 16)), full((1, 16)),
            full((16, 8)), full((1, 8)),
            full((1, 8)), full((1, 1)),
        ],
        out_specs=pl.BlockSpec((_BB, 1), lambda i: (i, 0)),
        out_shape=jax.ShapeDtypeStruct((B, 1), jnp.float32),
    )(gu, gi, w1u, w1i, b1.reshape(1, 32),
      W2.T, b2.reshape(1, 16), W3.T, b3.reshape(1, 8), W4, b4.reshape(1, 1))
    return out.reshape(-1)


def kernel(user_ids, item_ids, user_table, item_table,
           W1, b1, W2, b2, W3, b3, W4, b4):
    u_ids = user_ids.astype(jnp.int32).reshape(NW * CHUNKS, IDX_CHUNK)
    i_ids = item_ids.astype(jnp.int32).reshape(NW * CHUNKS, IDX_CHUNK)
    c_tab = _combine_tables(user_table.T, item_table.T)
    gu, gi = _make_sc_gather()(u_ids, i_ids, c_tab)
    return _mlp(gu, gi, W1, b1, W2, b2, W3, b3, W4, b4)


# speedup vs baseline: 2.3353x; 1.1347x over previous
import jax
import jax.numpy as jnp
from jax import lax
from jax.experimental import pallas as pl
from jax.experimental.pallas import tpu as pltpu
from jax.experimental.pallas import tpu_sc as plsc

B = 16384
D = 64
PACK = 2 * D
NROWS = 1000000
_INFO = plsc.get_sparse_core_info()
NC = _INFO.num_cores
NS = _INFO.num_subcores
NW = NC * NS
B_PER_W = B // NW
IDX_CHUNK = 128
CHUNKS = B_PER_W // IDX_CHUNK

_TC = 8192


def _combine_body(u_ref, i_ref, c_ref):
    c_ref[...] = jnp.concatenate([u_ref[...].T, i_ref[...].T], axis=1)


def _combine_tables(u_tabT, i_tabT):
    grid = (NROWS + _TC - 1) // _TC
    return pl.pallas_call(
        _combine_body,
        grid=(grid,),
        in_specs=[
            pl.BlockSpec((D, _TC), lambda g: (0, g)),
            pl.BlockSpec((D, _TC), lambda g: (0, g)),
        ],
        out_specs=pl.BlockSpec((_TC, PACK), lambda g: (g, 0)),
        out_shape=jax.ShapeDtypeStruct((NROWS, PACK), jnp.float32),
    )(u_tabT, i_tabT)


def _sc_gather_body(u_ids, i_ids, c_tab, u_out, i_out, idx_v, rows_v, sem):
    wid = lax.axis_index("s") * NC + lax.axis_index("c")
    row0 = wid * CHUNKS
    base = wid * B_PER_W
    for ids, out in ((u_ids, u_out), (i_ids, i_out)):
        pltpu.sync_copy(ids.at[pl.ds(row0, CHUNKS)], idx_v)
        copies = []
        for j in range(CHUNKS):
            copies.append(pltpu.async_copy(
                c_tab.at[idx_v.at[j]],
                rows_v.at[pl.ds(j * IDX_CHUNK, IDX_CHUNK)], sem))
        for c in copies:
            c.wait()
        pltpu.sync_copy(rows_v, out.at[pl.ds(base, B_PER_W)])


def _make_sc_gather():
    mesh = plsc.VectorSubcoreMesh(core_axis_name="c", subcore_axis_name="s")
    return pl.kernel(
        _sc_gather_body,
        mesh=mesh,
        out_type=[
            jax.ShapeDtypeStruct((B, PACK), jnp.float32),
            jax.ShapeDtypeStruct((B, PACK), jnp.float32),
        ],
        scratch_types=[
            pltpu.VMEM((CHUNKS, IDX_CHUNK), jnp.int32),
            pltpu.VMEM((B_PER_W, PACK), jnp.float32),
            pltpu.SemaphoreType.DMA,
        ],
    )


_BB = 2048


def _mlp_body(u_ref, i_ref, w1u_ref, w1i_ref, b1_ref, w2_ref, b2_ref,
              w3_ref, b3_ref, w4_ref, b4_ref, out_ref):
    h = jnp.dot(u_ref[...], w1u_ref[...], preferred_element_type=jnp.float32)
    h = h + jnp.dot(i_ref[...], w1i_ref[...],
                    preferred_element_type=jnp.float32)
    h = jnp.maximum(h + b1_ref[...], 0.0)
    h = jnp.maximum(jnp.dot(h, w2_ref[...], preferred_element_type=jnp.float32)
                    + b2_ref[...], 0.0)
    h = jnp.maximum(jnp.dot(h, w3_ref[...], preferred_element_type=jnp.float32)
                    + b3_ref[...], 0.0)
    out_ref[...] = (jnp.sum(h * w4_ref[...], axis=1, keepdims=True)
                    + b4_ref[...])


def _mlp(gu, gi, W1, b1, W2, b2, W3, b3, W4, b4):
    z = jnp.zeros((D, 32), jnp.float32)
    w1u = jnp.concatenate([W1[:, :D].T, z], axis=0)
    w1i = jnp.concatenate([z, W1[:, D:].T], axis=0)
    grid = B // _BB
    full = lambda s: pl.BlockSpec(s, lambda i: (0, 0))
    blk = lambda w: pl.BlockSpec((_BB, w), lambda i: (i, 0))
    out = pl.pallas_call(
        _mlp_body,
        grid=(grid,),
        in_specs=[
            blk(PACK), blk(PACK),
            full((PACK, 32)), full((PACK, 32)), full((1, 32)),
            full((32, 16)), full((1, 16)),
            full((16, 8)), full((1, 8)),
            full((1, 8)), full((1, 1)),
        ],
        out_specs=pl.BlockSpec((_BB, 1), lambda i: (i, 0)),
        out_shape=jax.ShapeDtypeStruct((B, 1), jnp.float32),
    )(gu, gi, w1u, w1i, b1.reshape(1, 32),
      W2.T, b2.reshape(1, 16), W3.T, b3.reshape(1, 8), W4, b4.reshape(1, 1))
    return out.reshape(-1)


def kernel(user_ids, item_ids, user_table, item_table,
           W1, b1, W2, b2, W3, b3, W4, b4):
    u_ids = user_ids.astype(jnp.int32).reshape(NW * CHUNKS, IDX_CHUNK)
    i_ids = item_ids.astype(jnp.int32).reshape(NW * CHUNKS, IDX_CHUNK)
    c_tab = _combine_tables(user_table.T, item_table.T)
    gu, gi = _make_sc_gather()(u_ids, i_ids, c_tab)
    return _mlp(gu, gi, W1, b1, W2, b2, W3, b3, W4, b4)


# speedup vs baseline: 2.4892x; 1.0659x over previous
import jax
import jax.numpy as jnp
from jax import lax
from jax.experimental import pallas as pl
from jax.experimental.pallas import tpu as pltpu
from jax.experimental.pallas import tpu_sc as plsc

B = 16384
D = 64
PACK = 2 * D
NROWS = 1000000
_INFO = plsc.get_sparse_core_info()
NC = _INFO.num_cores
NS = _INFO.num_subcores
NW = NC * NS
B_PER_W = B // NW
IDX_CHUNK = 128
CHUNKS = B_PER_W // IDX_CHUNK

_TC = 16384


def _combine_body(u_ref, i_ref, c_ref):
    c_ref[...] = jnp.concatenate([u_ref[...].T, i_ref[...].T], axis=1)


def _combine_tables(u_tabT, i_tabT):
    grid = (NROWS + _TC - 1) // _TC
    return pl.pallas_call(
        _combine_body,
        grid=(grid,),
        in_specs=[
            pl.BlockSpec((D, _TC), lambda g: (0, g)),
            pl.BlockSpec((D, _TC), lambda g: (0, g)),
        ],
        out_specs=pl.BlockSpec((_TC, PACK), lambda g: (g, 0)),
        out_shape=jax.ShapeDtypeStruct((NROWS, PACK), jnp.float32),
    )(u_tabT, i_tabT)


def _sc_gather_body(u_ids, i_ids, c_tab, u_out, i_out, idx_v, rows_v, sem):
    wid = lax.axis_index("s") * NC + lax.axis_index("c")
    row0 = wid * CHUNKS
    base = wid * B_PER_W
    for ids, out in ((u_ids, u_out), (i_ids, i_out)):
        pltpu.sync_copy(ids.at[pl.ds(row0, CHUNKS)], idx_v)
        copies = []
        for j in range(CHUNKS):
            copies.append(pltpu.async_copy(
                c_tab.at[idx_v.at[j]],
                rows_v.at[pl.ds(j * IDX_CHUNK, IDX_CHUNK)], sem))
        for c in copies:
            c.wait()
        pltpu.sync_copy(rows_v, out.at[pl.ds(base, B_PER_W)])


def _make_sc_gather():
    mesh = plsc.VectorSubcoreMesh(core_axis_name="c", subcore_axis_name="s")
    return pl.kernel(
        _sc_gather_body,
        mesh=mesh,
        out_type=[
            jax.ShapeDtypeStruct((B, PACK), jnp.float32),
            jax.ShapeDtypeStruct((B, PACK), jnp.float32),
        ],
        scratch_types=[
            pltpu.VMEM((CHUNKS, IDX_CHUNK), jnp.int32),
            pltpu.VMEM((B_PER_W, PACK), jnp.float32),
            pltpu.SemaphoreType.DMA,
        ],
    )


_BB = 2048


def _mlp_body(u_ref, i_ref, w1u_ref, w1i_ref, b1_ref, w2_ref, b2_ref,
              w3_ref, b3_ref, w4_ref, b4_ref, out_ref):
    h = jnp.dot(u_ref[...], w1u_ref[...], preferred_element_type=jnp.float32)
    h = h + jnp.dot(i_ref[...], w1i_ref[...],
                    preferred_element_type=jnp.float32)
    h = jnp.maximum(h + b1_ref[...], 0.0)
    h = jnp.maximum(jnp.dot(h, w2_ref[...], preferred_element_type=jnp.float32)
                    + b2_ref[...], 0.0)
    h = jnp.maximum(jnp.dot(h, w3_ref[...], preferred_element_type=jnp.float32)
                    + b3_ref[...], 0.0)
    out_ref[...] = (jnp.sum(h * w4_ref[...], axis=1, keepdims=True)
                    + b4_ref[...])


def _mlp(gu, gi, W1, b1, W2, b2, W3, b3, W4, b4):
    z = jnp.zeros((D, 32), jnp.float32)
    w1u = jnp.concatenate([W1[:, :D].T, z], axis=0)
    w1i = jnp.concatenate([z, W1[:, D:].T], axis=0)
    grid = B // _BB
    full = lambda s: pl.BlockSpec(s, lambda i: (0, 0))
    blk = lambda w: pl.BlockSpec((_BB, w), lambda i: (i, 0))
    out = pl.pallas_call(
        _mlp_body,
        grid=(grid,),
        in_specs=[
            blk(PACK), blk(PACK),
            full((PACK, 32)), full((PACK, 32)), full((1, 32)),
            full((32, 16)), full((1, 16)),
            full((16, 8)), full((1, 8)),
            full((1, 8)), full((1, 1)),
        ],
        out_specs=pl.BlockSpec((_BB, 1), lambda i: (i, 0)),
        out_shape=jax.ShapeDtypeStruct((B, 1), jnp.float32),
    )(gu, gi, w1u, w1i, b1.reshape(1, 32),
      W2.T, b2.reshape(1, 16), W3.T, b3.reshape(1, 8), W4, b4.reshape(1, 1))
    return out.reshape(-1)


def kernel(user_ids, item_ids, user_table, item_table,
           W1, b1, W2, b2, W3, b3, W4, b4):
    u_ids = user_ids.astype(jnp.int32).reshape(NW * CHUNKS, IDX_CHUNK)
    i_ids = item_ids.astype(jnp.int32).reshape(NW * CHUNKS, IDX_CHUNK)
    c_tab = _combine_tables(user_table.T, item_table.T)
    gu, gi = _make_sc_gather()(u_ids, i_ids, c_tab)
    return _mlp(gu, gi, W1, b1, W2, b2, W3, b3, W4, b4)


# speedup vs baseline: 3.5542x; 1.4278x over previous
import jax
import jax.numpy as jnp
from jax import lax
from jax.experimental import pallas as pl
from jax.experimental.pallas import tpu as pltpu
from jax.experimental.pallas import tpu_sc as plsc

B = 16384
D = 64
PACK = 2 * D
NROWS = 1000000
_INFO = plsc.get_sparse_core_info()
NC = _INFO.num_cores
NS = _INFO.num_subcores
NW = NC * NS
B_PER_W = B // NW
IDX_CHUNK = 128
CHUNKS = B_PER_W // IDX_CHUNK

_TC = 16384


def _combine_body(u_ref, i_ref, c_ref):
    c = jnp.concatenate([u_ref[...].T, i_ref[...].T], axis=1)
    c_ref[...] = pltpu.bitcast(c.astype(jnp.bfloat16), jnp.int32)


def _combine_tables(u_tabT, i_tabT):
    grid = (NROWS + _TC - 1) // _TC
    return pl.pallas_call(
        _combine_body,
        grid=(grid,),
        in_specs=[
            pl.BlockSpec((D, _TC), lambda g: (0, g)),
            pl.BlockSpec((D, _TC), lambda g: (0, g)),
        ],
        out_specs=pl.BlockSpec((_TC // 2, PACK), lambda g: (g, 0)),
        out_shape=jax.ShapeDtypeStruct((NROWS // 2, PACK), jnp.int32),
    )(u_tabT, i_tabT)


def _sc_gather_body(u_ids, i_ids, c_tab, u_out, i_out, idx_v, rows_v, sem):
    wid = lax.axis_index("s") * NC + lax.axis_index("c")
    row0 = wid * CHUNKS
    base = wid * B_PER_W
    for ids, out in ((u_ids, u_out), (i_ids, i_out)):
        pltpu.sync_copy(ids.at[pl.ds(row0, CHUNKS)], idx_v)
        copies = []
        for j in range(CHUNKS):
            copies.append(pltpu.async_copy(
                c_tab.at[idx_v.at[j]],
                rows_v.at[pl.ds(j * IDX_CHUNK, IDX_CHUNK)], sem))
        for c in copies:
            c.wait()
        pltpu.sync_copy(rows_v, out.at[pl.ds(base, B_PER_W)])


def _make_sc_gather():
    mesh = plsc.VectorSubcoreMesh(core_axis_name="c", subcore_axis_name="s")
    return pl.kernel(
        _sc_gather_body,
        mesh=mesh,
        out_type=[
            jax.ShapeDtypeStruct((B, PACK), jnp.int32),
            jax.ShapeDtypeStruct((B, PACK), jnp.int32),
        ],
        scratch_types=[
            pltpu.VMEM((CHUNKS, IDX_CHUNK), jnp.int32),
            pltpu.VMEM((B_PER_W, PACK), jnp.int32),
            pltpu.SemaphoreType.DMA,
        ],
    )


_BB = 2048


def _mlp_body(u_ref, i_ref, pu_ref, pi_ref, w1u_ref, w1i_ref, b1_ref,
              w2_ref, b2_ref, w3_ref, b3_ref, w4_ref, b4_ref, out_ref):
    ug = u_ref[...]
    ig = i_ref[...]
    u = lax.bitcast_convert_type(
        jnp.where(pu_ref[...] > 0, ug & jnp.int32(-65536), ug << 16),
        jnp.float32)
    i = lax.bitcast_convert_type(
        jnp.where(pi_ref[...] > 0, ig & jnp.int32(-65536), ig << 16),
        jnp.float32)
    h = jnp.dot(u, w1u_ref[...], preferred_element_type=jnp.float32)
    h = h + jnp.dot(i, w1i_ref[...], preferred_element_type=jnp.float32)
    h = jnp.maximum(h + b1_ref[...], 0.0)
    h = jnp.maximum(jnp.dot(h, w2_ref[...], preferred_element_type=jnp.float32)
                    + b2_ref[...], 0.0)
    h = jnp.maximum(jnp.dot(h, w3_ref[...], preferred_element_type=jnp.float32)
                    + b3_ref[...], 0.0)
    out_ref[...] = (jnp.sum(h * w4_ref[...], axis=1, keepdims=True)
                    + b4_ref[...])


def _mlp(gu, gi, pu, pi, W1, b1, W2, b2, W3, b3, W4, b4):
    z = jnp.zeros((D, 32), jnp.float32)
    w1u = jnp.concatenate([W1[:, :D].T, z], axis=0)
    w1i = jnp.concatenate([z, W1[:, D:].T], axis=0)
    grid = B // _BB
    full = lambda s: pl.BlockSpec(s, lambda i: (0, 0))
    blk = lambda w: pl.BlockSpec((_BB, w), lambda i: (i, 0))
    out = pl.pallas_call(
        _mlp_body,
        grid=(grid,),
        in_specs=[
            blk(PACK), blk(PACK), blk(1), blk(1),
            full((PACK, 32)), full((PACK, 32)), full((1, 32)),
            full((32, 16)), full((1, 16)),
            full((16, 8)), full((1, 8)),
            full((1, 8)), full((1, 1)),
        ],
        out_specs=pl.BlockSpec((_BB, 1), lambda i: (i, 0)),
        out_shape=jax.ShapeDtypeStruct((B, 1), jnp.float32),
    )(gu, gi, pu, pi, w1u, w1i, b1.reshape(1, 32),
      W2.T, b2.reshape(1, 16), W3.T, b3.reshape(1, 8), W4, b4.reshape(1, 1))
    return out.reshape(-1)


def kernel(user_ids, item_ids, user_table, item_table,
           W1, b1, W2, b2, W3, b3, W4, b4):
    uid = user_ids.astype(jnp.int32)
    iid = item_ids.astype(jnp.int32)
    u_ids = (uid // 2).reshape(NW * CHUNKS, IDX_CHUNK)
    i_ids = (iid // 2).reshape(NW * CHUNKS, IDX_CHUNK)
    pu = (uid % 2).reshape(B, 1)
    pi = (iid % 2).reshape(B, 1)
    c_tab = _combine_tables(user_table.T, item_table.T)
    gu, gi = _make_sc_gather()(u_ids, i_ids, c_tab)
    return _mlp(gu, gi, pu, pi, W1, b1, W2, b2, W3, b3, W4, b4)


# speedup vs baseline: 3.6916x; 1.0387x over previous
import jax
import jax.numpy as jnp
from jax import lax
from jax.experimental import pallas as pl
from jax.experimental.pallas import tpu as pltpu
from jax.experimental.pallas import tpu_sc as plsc

B = 16384
D = 64
PACK = 2 * D
NROWS = 1000000
_INFO = plsc.get_sparse_core_info()
NC = _INFO.num_cores
NS = _INFO.num_subcores
NW = NC * NS
B_PER_W = B // NW
IDX_CHUNK = 128
CHUNKS = B_PER_W // IDX_CHUNK

_TC = 24576


def _combine_body(u_ref, i_ref, c_ref):
    c = jnp.concatenate([u_ref[...].T, i_ref[...].T], axis=1)
    c_ref[...] = pltpu.bitcast(c.astype(jnp.bfloat16), jnp.int32)


def _combine_tables(u_tabT, i_tabT):
    grid = (NROWS + _TC - 1) // _TC
    return pl.pallas_call(
        _combine_body,
        grid=(grid,),
        in_specs=[
            pl.BlockSpec((D, _TC), lambda g: (0, g)),
            pl.BlockSpec((D, _TC), lambda g: (0, g)),
        ],
        out_specs=pl.BlockSpec((_TC // 2, PACK), lambda g: (g, 0)),
        out_shape=jax.ShapeDtypeStruct((NROWS // 2, PACK), jnp.int32),
    )(u_tabT, i_tabT)


def _sc_gather_body(u_ids, i_ids, c_tab, u_out, i_out, idx_v, rows_v, sem):
    wid = lax.axis_index("s") * NC + lax.axis_index("c")
    row0 = wid * CHUNKS
    base = wid * B_PER_W
    for ids, out in ((u_ids, u_out), (i_ids, i_out)):
        pltpu.sync_copy(ids.at[pl.ds(row0, CHUNKS)], idx_v)
        copies = []
        for j in range(CHUNKS):
            copies.append(pltpu.async_copy(
                c_tab.at[idx_v.at[j]],
                rows_v.at[pl.ds(j * IDX_CHUNK, IDX_CHUNK)], sem))
        for c in copies:
            c.wait()
        pltpu.sync_copy(rows_v, out.at[pl.ds(base, B_PER_W)])


def _make_sc_gather():
    mesh = plsc.VectorSubcoreMesh(core_axis_name="c", subcore_axis_name="s")
    return pl.kernel(
        _sc_gather_body,
        mesh=mesh,
        out_type=[
            jax.ShapeDtypeStruct((B, PACK), jnp.int32),
            jax.ShapeDtypeStruct((B, PACK), jnp.int32),
        ],
        scratch_types=[
            pltpu.VMEM((CHUNKS, IDX_CHUNK), jnp.int32),
            pltpu.VMEM((B_PER_W, PACK), jnp.int32),
            pltpu.SemaphoreType.DMA,
        ],
    )


_BB = 2048


def _mlp_body(u_ref, i_ref, pu_ref, pi_ref, w1u_ref, w1i_ref, b1_ref,
              w2_ref, b2_ref, w3_ref, b3_ref, w4_ref, b4_ref, out_ref):
    ug = u_ref[...]
    ig = i_ref[...]
    u = lax.bitcast_convert_type(
        jnp.where(pu_ref[...] > 0, ug & jnp.int32(-65536), ug << 16),
        jnp.float32)
    i = lax.bitcast_convert_type(
        jnp.where(pi_ref[...] > 0, ig & jnp.int32(-65536), ig << 16),
        jnp.float32)
    h = jnp.dot(u, w1u_ref[...], preferred_element_type=jnp.float32)
    h = h + jnp.dot(i, w1i_ref[...], preferred_element_type=jnp.float32)
    h = jnp.maximum(h + b1_ref[...], 0.0)
    h = jnp.maximum(jnp.dot(h, w2_ref[...], preferred_element_type=jnp.float32)
                    + b2_ref[...], 0.0)
    h = jnp.maximum(jnp.dot(h, w3_ref[...], preferred_element_type=jnp.float32)
                    + b3_ref[...], 0.0)
    out_ref[...] = (jnp.sum(h * w4_ref[...], axis=1, keepdims=True)
                    + b4_ref[...])


def _mlp(gu, gi, pu, pi, W1, b1, W2, b2, W3, b3, W4, b4):
    z = jnp.zeros((D, 32), jnp.float32)
    w1u = jnp.concatenate([W1[:, :D].T, z], axis=0)
    w1i = jnp.concatenate([z, W1[:, D:].T], axis=0)
    grid = B // _BB
    full = lambda s: pl.BlockSpec(s, lambda i: (0, 0))
    blk = lambda w: pl.BlockSpec((_BB, w), lambda i: (i, 0))
    out = pl.pallas_call(
        _mlp_body,
        grid=(grid,),
        in_specs=[
            blk(PACK), blk(PACK), blk(1), blk(1),
            full((PACK, 32)), full((PACK, 32)), full((1, 32)),
            full((32, 16)), full((1, 16)),
            full((16, 8)), full((1, 8)),
            full((1, 8)), full((1, 1)),
        ],
        out_specs=pl.BlockSpec((_BB, 1), lambda i: (i, 0)),
        out_shape=jax.ShapeDtypeStruct((B, 1), jnp.float32),
    )(gu, gi, pu, pi, w1u, w1i, b1.reshape(1, 32),
      W2.T, b2.reshape(1, 16), W3.T, b3.reshape(1, 8), W4, b4.reshape(1, 1))
    return out.reshape(-1)


def kernel(user_ids, item_ids, user_table, item_table,
           W1, b1, W2, b2, W3, b3, W4, b4):
    uid = user_ids.astype(jnp.int32)
    iid = item_ids.astype(jnp.int32)
    u_ids = (uid // 2).reshape(NW * CHUNKS, IDX_CHUNK)
    i_ids = (iid // 2).reshape(NW * CHUNKS, IDX_CHUNK)
    pu = (uid % 2).reshape(B, 1)
    pi = (iid % 2).reshape(B, 1)
    c_tab = _combine_tables(user_table.T, item_table.T)
    gu, gi = _make_sc_gather()(u_ids, i_ids, c_tab)
    return _mlp(gu, gi, pu, pi, W1, b1, W2, b2, W3, b3, W4, b4)
